# Initial kernel scaffold; baseline (speedup 1.0000x reference)
import jax
import jax.numpy as jnp
from jax.experimental import pallas as pl


def kernel(x, emb, W1, b1, W2, b2, W3, b3):
    raise NotImplementedError("write your pallas kernel here")


# speedup vs baseline: 2.9886x; 2.9886x over previous
import functools

import jax
import jax.numpy as jnp
from jax import lax
from jax.experimental import pallas as pl
from jax.experimental.pallas import tpu as pltpu
from jax.experimental.pallas import tpu_sc as plsc

VOCAB = 100000
EMB_DIM = 128
HIDDEN = 512
OUT_DIM = 128
BATCH = 4096
SEQ = 200

NTOK = BATCH * SEQ
NC, NS = 2, 16
NW = NC * NS
IDX_ROWS = NTOK // 128
ROWS_PER_W = IDX_ROWS // NW
K = 4
GROUPS = ROWS_PER_W // K


@functools.partial(
    pl.kernel,
    mesh=plsc.VectorSubcoreMesh(core_axis_name="c", subcore_axis_name="s"),
    out_type=jax.ShapeDtypeStruct((NTOK, EMB_DIM), jnp.float32),
    scratch_types=[
        pltpu.VMEM((K, 128), jnp.int32),
        pltpu.VMEM((K * 128, EMB_DIM), jnp.float32),
        pltpu.SemaphoreType.DMA,
    ],
)
def _sc_gather(x2_hbm, emb_hbm, out_hbm, idx_v, rows_v, sem):
    wid = lax.axis_index("s") * NC + lax.axis_index("c")
    row0 = wid * ROWS_PER_W

    def group(g, carry):
        r = row0 + g * K
        pltpu.sync_copy(x2_hbm.at[pl.ds(r, K)], idx_v)
        for j in range(K):
            pltpu.async_copy(
                emb_hbm.at[idx_v.at[j]], rows_v.at[pl.ds(j * 128, 128)], sem
            )
        for j in range(K):
            pltpu.make_async_copy(
                emb_hbm.at[idx_v.at[j]], rows_v.at[pl.ds(j * 128, 128)], sem
            ).wait()
        pltpu.sync_copy(rows_v, out_hbm.at[pl.ds(r * 128, K * 128)])
        return carry

    lax.fori_loop(0, GROUPS, group, 0)


def _mlp_body(e_ref, w1_ref, b1_ref, w2_ref, b2_ref, w3_ref, b3_ref, o_ref):
    h = jnp.dot(e_ref[...], w1_ref[...], preferred_element_type=jnp.float32)
    h = jnp.maximum(h + b1_ref[...], 0.0)
    h = jnp.dot(h, w2_ref[...], preferred_element_type=jnp.float32)
    h = jnp.maximum(h + b2_ref[...], 0.0)
    o = jnp.dot(h, w3_ref[...], preferred_element_type=jnp.float32)
    o_ref[...] = o + b3_ref[...]


TILE = 1024


def _mlp(e, W1, b1, W2, b2, W3, b3):
    return pl.pallas_call(
        _mlp_body,
        grid=(NTOK // TILE,),
        in_specs=[
            pl.BlockSpec((TILE, EMB_DIM), lambda i: (i, 0)),
            pl.BlockSpec((EMB_DIM, HIDDEN), lambda i: (0, 0)),
            pl.BlockSpec((1, HIDDEN), lambda i: (0, 0)),
            pl.BlockSpec((HIDDEN, HIDDEN), lambda i: (0, 0)),
            pl.BlockSpec((1, HIDDEN), lambda i: (0, 0)),
            pl.BlockSpec((HIDDEN, OUT_DIM), lambda i: (0, 0)),
            pl.BlockSpec((1, OUT_DIM), lambda i: (0, 0)),
        ],
        out_specs=pl.BlockSpec((TILE, OUT_DIM), lambda i: (i, 0)),
        out_shape=jax.ShapeDtypeStruct((NTOK, OUT_DIM), jnp.float32),
        compiler_params=pltpu.CompilerParams(
            dimension_semantics=("arbitrary",)
        ),
    )(e, W1, b1.reshape(1, HIDDEN), W2, b2.reshape(1, HIDDEN),
      W3, b3.reshape(1, OUT_DIM))


def kernel(x, emb, W1, b1, W2, b2, W3, b3):
    x2 = x.reshape(IDX_ROWS, 128).astype(jnp.int32)
    e = _sc_gather(x2, emb)
    out = _mlp(e, W1, b1, W2, b2, W3, b3)
    return out.reshape(BATCH, SEQ, OUT_DIM)


# speedup vs baseline: 3.4378x; 1.1503x over previous
import functools

import jax
import jax.numpy as jnp
from jax import lax
from jax.experimental import pallas as pl
from jax.experimental.pallas import tpu as pltpu
from jax.experimental.pallas import tpu_sc as plsc

VOCAB = 100000
EMB_DIM = 128
HIDDEN = 512
OUT_DIM = 128
BATCH = 4096
SEQ = 200

NTOK = BATCH * SEQ
NC, NS = 2, 16
NW = NC * NS
IDX_ROWS = NTOK // 128
ROWS_PER_W = IDX_ROWS // NW
K = 4
GROUPS = ROWS_PER_W // K


@functools.partial(
    pl.kernel,
    mesh=plsc.VectorSubcoreMesh(core_axis_name="c", subcore_axis_name="s"),
    out_type=jax.ShapeDtypeStruct((NTOK, EMB_DIM), jnp.float32),
    scratch_types=[
        pltpu.VMEM((K, 128), jnp.int32),
        pltpu.VMEM((K * 128, EMB_DIM), jnp.float32),
        pltpu.SemaphoreType.DMA,
    ],
)
def _sc_gather(x2_hbm, emb_hbm, out_hbm, idx_v, rows_v, sem):
    wid = lax.axis_index("s") * NC + lax.axis_index("c")
    row0 = wid * ROWS_PER_W

    def group(g, carry):
        r = row0 + g * K
        pltpu.sync_copy(x2_hbm.at[pl.ds(r, K)], idx_v)
        for j in range(K):
            pltpu.async_copy(
                emb_hbm.at[idx_v.at[j]], rows_v.at[pl.ds(j * 128, 128)], sem
            )
        for j in range(K):
            pltpu.make_async_copy(
                emb_hbm.at[idx_v.at[j]], rows_v.at[pl.ds(j * 128, 128)], sem
            ).wait()
        pltpu.sync_copy(rows_v, out_hbm.at[pl.ds(r * 128, K * 128)])
        return carry

    lax.fori_loop(0, GROUPS, group, 0)


def _mlp_body(e_ref, w1_ref, b1_ref, w2_ref, b2_ref, w3_ref, b3_ref, o_ref):
    h = jnp.dot(e_ref[...].astype(jnp.bfloat16), w1_ref[...],
                preferred_element_type=jnp.float32).astype(jnp.bfloat16)
    h = jnp.maximum(h + b1_ref[...], jnp.bfloat16(0.0))
    h = jnp.dot(h, w2_ref[...],
                preferred_element_type=jnp.float32).astype(jnp.bfloat16)
    h = jnp.maximum(h + b2_ref[...], jnp.bfloat16(0.0))
    o = jnp.dot(h, w3_ref[...], preferred_element_type=jnp.float32)
    o_ref[...] = o + b3_ref[...]


TILE = 4096


def _mlp(e, W1, b1, W2, b2, W3, b3):
    return pl.pallas_call(
        _mlp_body,
        grid=(NTOK // TILE,),
        in_specs=[
            pl.BlockSpec((TILE, EMB_DIM), lambda i: (i, 0)),
            pl.BlockSpec((EMB_DIM, HIDDEN), lambda i: (0, 0)),
            pl.BlockSpec((1, HIDDEN), lambda i: (0, 0)),
            pl.BlockSpec((HIDDEN, HIDDEN), lambda i: (0, 0)),
            pl.BlockSpec((1, HIDDEN), lambda i: (0, 0)),
            pl.BlockSpec((HIDDEN, OUT_DIM), lambda i: (0, 0)),
            pl.BlockSpec((1, OUT_DIM), lambda i: (0, 0)),
        ],
        out_specs=pl.BlockSpec((TILE, OUT_DIM), lambda i: (i, 0)),
        out_shape=jax.ShapeDtypeStruct((NTOK, OUT_DIM), jnp.float32),
        compiler_params=pltpu.CompilerParams(
            dimension_semantics=("arbitrary",)
        ),
    )(e, W1, b1.reshape(1, HIDDEN), W2, b2.reshape(1, HIDDEN),
      W3, b3.reshape(1, OUT_DIM))


def kernel(x, emb, W1, b1, W2, b2, W3, b3):
    x2 = x.reshape(IDX_ROWS, 128).astype(jnp.int32)
    e = _sc_gather(x2, emb)
    out = _mlp(e, W1.astype(jnp.bfloat16), b1.astype(jnp.bfloat16),
               W2.astype(jnp.bfloat16), b2.astype(jnp.bfloat16),
               W3.astype(jnp.bfloat16), b3)
    return out.reshape(BATCH, SEQ, OUT_DIM)
